# baseline (speedup 1.0000x reference)
import functools

import jax
import jax.numpy as jnp
from jax import lax
from jax.experimental import pallas as pl
from jax.experimental.pallas import tpu as pltpu
from jax.experimental.pallas import tpu_sc as plsc

MAXLEN = 200
EMBED = 64
BATCH = 4096
NC, NS = 2, 16
NW = NC * NS
COLS_W = BATCH // NW
NBUF = 2
LANES = 16
NG = COLS_W // LANES


def _body(x_hbm, tok_hbm, pos_hbm, out_hbm,
          idx_v, pos_v, hoff, gbuf, sbuf, gsem, ssem):
  wid = lax.axis_index("s") * NC + lax.axis_index("c")
  b0 = wid * COLS_W

  pltpu.sync_copy(x_hbm.at[:, pl.ds(b0, COLS_W)], idx_v)
  pltpu.sync_copy(pos_hbm, pos_v)

  @pl.loop(0, MAXLEN)
  def _(l):
    for g in range(NG):
      sl = pl.ds(g * LANES, LANES)
      v = idx_v[l, sl]
      idx_v[l, sl] = lax.shift_right_logical(v, 1)
      hoff[l, sl] = lax.shift_left((v & 1), 6)

  def g_start(l, s):
    pltpu.async_copy(tok_hbm.at[idx_v.at[l]], gbuf.at[s], gsem.at[s])

  def g_wait(l, s):
    pltpu.make_async_copy(tok_hbm.at[idx_v.at[l]], gbuf.at[s],
                          gsem.at[s]).wait()

  def s_start(l, s):
    pltpu.async_copy(sbuf.at[s], out_hbm.at[l, :, pl.ds(b0, COLS_W)],
                     ssem.at[s])

  def s_wait(l, s):
    pltpu.make_async_copy(sbuf.at[s], out_hbm.at[l, :, pl.ds(b0, COLS_W)],
                          ssem.at[s]).wait()

  def compute(l, s):
    gref = gbuf.at[s]

    @pl.loop(0, EMBED // LANES)
    def _(eg):
      pvec = pos_v[l, pl.ds(eg * LANES, LANES)]
      for ei in range(LANES):
        e = eg * LANES + ei
        ps = jnp.broadcast_to(pvec[ei], (LANES,))
        for g in range(NG):
          sl = pl.ds(g * LANES, LANES)
          rid = lax.iota(jnp.int32, LANES) + (g * LANES)
          val = plsc.load_gather(gref, [rid, hoff[l, sl] + e])
          sbuf[s, e, sl] = val + ps

  for s in range(NBUF - 1):
    g_start(s, s)

  @pl.loop(0, MAXLEN // NBUF)
  def _(t):
    for b in range(NBUF):
      l = t * NBUF + b
      bn = (b + NBUF - 1) % NBUF
      @pl.when(l + NBUF - 1 < MAXLEN)
      def _():
        @pl.when(l >= 1)
        def _():
          s_wait(l - 1, bn)
        g_start(l + NBUF - 1, bn)

      g_wait(l, b)
      compute(l, b)
      s_start(l, b)

  for b in range(NBUF):
    s_wait(MAXLEN - NBUF + b, b)


@functools.partial(
    pl.kernel,
    out_type=jax.ShapeDtypeStruct((MAXLEN, EMBED, BATCH), jnp.float32),
    mesh=plsc.VectorSubcoreMesh(
        core_axis_name="c", subcore_axis_name="s",
        num_cores=NC, num_subcores=NS),
    scratch_types=[
        pltpu.VMEM((MAXLEN, COLS_W), jnp.int32),
        pltpu.VMEM((MAXLEN, EMBED), jnp.float32),
        pltpu.VMEM((MAXLEN, COLS_W), jnp.int32),
        pltpu.VMEM((NBUF, COLS_W, 128), jnp.float32),
        pltpu.VMEM((NBUF, EMBED, COLS_W), jnp.float32),
        pltpu.SemaphoreType.DMA((NBUF,)),
        pltpu.SemaphoreType.DMA((NBUF,)),
    ],
    compiler_params=pltpu.CompilerParams(
        use_tc_tiling_on_sc=True, needs_layout_passes=False),
)
def _emb(x_hbm, tok_hbm, pos_hbm, out_hbm,
         idx_v, pos_v, hoff, gbuf, sbuf, gsem, ssem):
  _body(x_hbm, tok_hbm, pos_hbm, out_hbm,
        idx_v, pos_v, hoff, gbuf, sbuf, gsem, ssem)


@jax.jit
def kernel(x, token_table, pos_table):
  xt = jnp.swapaxes(x.astype(jnp.int32), 0, 1)
  tok2 = jnp.reshape(token_table, (500000, 128))
  res = _emb(xt, tok2, pos_table)
  return jnp.transpose(res, (2, 0, 1))


# speedup vs baseline: 2.1251x; 2.1251x over previous
import functools

import jax
import jax.numpy as jnp
from jax import lax
from jax.experimental import pallas as pl
from jax.experimental.pallas import tpu as pltpu
from jax.experimental.pallas import tpu_sc as plsc

MAXLEN = 200
EMBED = 64
BATCH = 4096
NC, NS = 2, 16
NW = NC * NS
ROWS_W = BATCH // NW
NBUF = 2
LANES = 16
G1 = 128
G2 = MAXLEN - G1
OROWS = MAXLEN * EMBED // 128


def _body(x_hbm, tok_hbm, pos_hbm, out_hbm,
          idx_v, pos_v, buf, sbuf, g1s, g2s, ssem):
  wid = lax.axis_index("s") * NC + lax.axis_index("c")
  rbase = wid * ROWS_W

  pltpu.sync_copy(x_hbm.at[pl.ds(rbase, ROWS_W)], idx_v)
  pltpu.sync_copy(pos_hbm, pos_v)

  def g_start(r, b):
    pltpu.async_copy(tok_hbm.at[idx_v.at[r, pl.ds(0, G1)]],
                     buf.at[b, pl.ds(0, G1)], g1s.at[b])
    pltpu.async_copy(tok_hbm.at[idx_v.at[r, pl.ds(G1, G2)]],
                     buf.at[b, pl.ds(G1, G2)], g2s.at[b])

  def g_wait(r, b):
    pltpu.make_async_copy(tok_hbm.at[idx_v.at[r, pl.ds(0, G1)]],
                          buf.at[b, pl.ds(0, G1)], g1s.at[b]).wait()
    pltpu.make_async_copy(tok_hbm.at[idx_v.at[r, pl.ds(G1, G2)]],
                          buf.at[b, pl.ds(G1, G2)], g2s.at[b]).wait()

  def _dst(r):
    return out_hbm.at[pl.ds((rbase + r) * OROWS, OROWS)]

  def s_start(r, b):
    pltpu.async_copy(sbuf.at[b], _dst(r), ssem.at[b])

  def s_wait(r, b):
    pltpu.make_async_copy(sbuf.at[b], _dst(r), ssem.at[b]).wait()

  for b in range(NBUF - 1):
    g_start(b, b)

  @pl.loop(0, ROWS_W // NBUF)
  def _(t):
    for b in range(NBUF):
      r = t * NBUF + b
      bn = (b + NBUF - 1) % NBUF
      @pl.when(r + NBUF - 1 < ROWS_W)
      def _():
        @pl.when(r >= 1)
        def _():
          s_wait(r - 1, bn)
        g_start(r + NBUF - 1, bn)

      g_wait(r, b)
      bref = buf.at[b]

      @pl.loop(0, MAXLEN // 8)
      def _(i):
        for rr in range(8):
          for j in range(EMBED // LANES):
            sl = pl.ds(j * LANES, LANES)
            ol = pl.ds((rr % 2) * EMBED + j * LANES, LANES)
            sbuf[b, i * 4 + rr // 2, ol] = (
                bref[i * 8 + rr, sl] + pos_v[i * 8 + rr, sl])

      s_start(r, b)

  for b in range(NBUF):
    s_wait(ROWS_W - NBUF + b, b)


@functools.partial(
    pl.kernel,
    out_type=jax.ShapeDtypeStruct((BATCH * OROWS, 128), jnp.float32),
    mesh=plsc.VectorSubcoreMesh(
        core_axis_name="c", subcore_axis_name="s",
        num_cores=NC, num_subcores=NS),
    scratch_types=[
        pltpu.VMEM((ROWS_W, MAXLEN), jnp.int32),
        pltpu.VMEM((MAXLEN, EMBED), jnp.float32),
        pltpu.VMEM((NBUF, MAXLEN, EMBED), jnp.float32),
        pltpu.VMEM((NBUF, OROWS, 128), jnp.float32),
        pltpu.SemaphoreType.DMA((NBUF,)),
        pltpu.SemaphoreType.DMA((NBUF,)),
        pltpu.SemaphoreType.DMA((NBUF,)),
    ],
    compiler_params=pltpu.CompilerParams(use_tc_tiling_on_sc=False),
)
def _emb(x_hbm, tok_hbm, pos_hbm, out_hbm,
         idx_v, pos_v, buf, sbuf, g1s, g2s, ssem):
  _body(x_hbm, tok_hbm, pos_hbm, out_hbm,
        idx_v, pos_v, buf, sbuf, g1s, g2s, ssem)


@jax.jit
def kernel(x, token_table, pos_table):
  out = _emb(x.astype(jnp.int32), token_table, pos_table)
  return jnp.reshape(out, (BATCH, MAXLEN, EMBED))
